# baseline (speedup 1.0000x reference)
import jax
import jax.numpy as jnp
from jax.experimental import pallas as pl


def _body(emb_ref, out_ref):
    s = emb_ref[0]
    half = pl.program_id(1)
    f, n = s.shape
    a = jnp.broadcast_to(s[:, None, :], (f, n, n))
    b = jnp.broadcast_to(s[:, :, None], (f, n, n))
    out_ref[0] = jnp.where(half == 0, a, b)


def kernel(x, row_embed, col_embed):
    bsz, _, h, w = x.shape
    f = row_embed.shape[1]
    emb = jnp.stack([col_embed[:w, :].T, row_embed[:h, :].T])
    out = pl.pallas_call(
        _body,
        grid=(bsz, 2),
        in_specs=[pl.BlockSpec((1, f, w), lambda b, c: (c, 0, 0))],
        out_specs=pl.BlockSpec((1, f, h, w), lambda b, c: (b, c, 0, 0)),
        out_shape=jax.ShapeDtypeStruct((bsz, 2 * f, h, w), jnp.float32),
    )(emb)
    return out


# speedup vs baseline: 1.0228x; 1.0228x over previous
import jax
import jax.numpy as jnp
from jax.experimental import pallas as pl
from jax.experimental.pallas import tpu as pltpu


def _body(emb_ref, out_ref, scratch_ref, sem):
    f, n = emb_ref.shape[1], emb_ref.shape[2]
    bsz = out_ref.shape[0]
    scratch_ref[:f] = jnp.broadcast_to(emb_ref[0][:, None, :], (f, n, n))
    scratch_ref[f:] = jnp.broadcast_to(emb_ref[1][:, :, None], (f, n, n))
    for b in range(bsz):
        pltpu.make_async_copy(scratch_ref, out_ref.at[b], sem.at[b]).start()
    for b in range(bsz):
        pltpu.make_async_copy(scratch_ref, out_ref.at[b], sem.at[b]).wait()


def kernel(x, row_embed, col_embed):
    bsz, _, h, w = x.shape
    f = row_embed.shape[1]
    emb = jnp.stack([col_embed[:w, :].T, row_embed[:h, :].T])
    out = pl.pallas_call(
        _body,
        in_specs=[pl.BlockSpec(memory_space=pltpu.MemorySpace.VMEM)],
        out_specs=pl.BlockSpec(memory_space=pl.ANY),
        out_shape=jax.ShapeDtypeStruct((bsz, 2 * f, h, w), jnp.float32),
        scratch_shapes=[
            pltpu.VMEM((2 * f, h, w), jnp.float32),
            pltpu.SemaphoreType.DMA((bsz,)),
        ],
    )(emb)
    return out


# speedup vs baseline: 1.5702x; 1.5351x over previous
import jax
import jax.numpy as jnp
from jax.experimental import pallas as pl
from jax.experimental.pallas import tpu as pltpu


def _body(etop_ref, ebot_ref, out_ref, scratch_ref, sem):
    f = etop_ref.shape[0]
    q = scratch_ref.shape[1]
    bsz = out_ref.shape[0]
    scratch_ref[:f] = jnp.broadcast_to(etop_ref[...][:, None, :], (f, q, 128))
    scratch_ref[f:, :, :64] = jnp.broadcast_to(ebot_ref[0][:, :, None], (f, q, 64))
    scratch_ref[f:, :, 64:] = jnp.broadcast_to(ebot_ref[1][:, :, None], (f, q, 64))
    for b in range(bsz):
        pltpu.make_async_copy(scratch_ref, out_ref.at[b], sem.at[b]).start()
    for b in range(bsz):
        pltpu.make_async_copy(scratch_ref, out_ref.at[b], sem.at[b]).wait()


def kernel(x, row_embed, col_embed):
    bsz, _, h, w = x.shape
    f = row_embed.shape[1]
    ct = col_embed[:w, :].T
    rt = row_embed[:h, :].T
    etop = jnp.concatenate([ct, ct], axis=1)
    ebot = jnp.stack([rt[:, 0::2], rt[:, 1::2]])
    out = pl.pallas_call(
        _body,
        in_specs=[
            pl.BlockSpec(memory_space=pltpu.MemorySpace.VMEM),
            pl.BlockSpec(memory_space=pltpu.MemorySpace.VMEM),
        ],
        out_specs=pl.BlockSpec(memory_space=pl.ANY),
        out_shape=jax.ShapeDtypeStruct((bsz, 2 * f, h * w // 128, 128), jnp.float32),
        scratch_shapes=[
            pltpu.VMEM((2 * f, h * w // 128, 128), jnp.float32),
            pltpu.SemaphoreType.DMA((bsz,)),
        ],
    )(etop, ebot)
    return out.reshape(bsz, 2 * f, h, w)


# speedup vs baseline: 1.5802x; 1.0064x over previous
import jax
import jax.numpy as jnp
from jax.experimental import pallas as pl
from jax.experimental.pallas import tpu as pltpu


def _body(etop_ref, ebot_ref, out_ref, s0, s1, s2, s3, sem):
    f = etop_ref.shape[0]
    scr = (s0, s1, s2, s3)
    q = s0.shape[1]
    fq = f // 2
    bsz = out_ref.shape[0]
    s0[:] = jnp.broadcast_to(etop_ref[:fq][:, None, :], (fq, q, 128))
    s1[:] = jnp.broadcast_to(etop_ref[fq:][:, None, :], (fq, q, 128))
    for k, s in ((0, s2), (1, s3)):
        s[:, :, :64] = jnp.broadcast_to(ebot_ref[0, k * fq:(k + 1) * fq][:, :, None], (fq, q, 64))
        s[:, :, 64:] = jnp.broadcast_to(ebot_ref[1, k * fq:(k + 1) * fq][:, :, None], (fq, q, 64))
    for b in range(bsz):
        for k in range(4):
            pltpu.make_async_copy(
                scr[k], out_ref.at[b, pl.ds(k * fq, fq)], sem.at[b, k]
            ).start()
    for b in range(bsz):
        for k in range(4):
            pltpu.make_async_copy(
                scr[k], out_ref.at[b, pl.ds(k * fq, fq)], sem.at[b, k]
            ).wait()


def kernel(x, row_embed, col_embed):
    bsz, _, h, w = x.shape
    f = row_embed.shape[1]
    ct = col_embed[:w, :].T
    rt = row_embed[:h, :].T
    etop = jnp.concatenate([ct, ct], axis=1)
    ebot = jnp.stack([rt[:, 0::2], rt[:, 1::2]])
    out = pl.pallas_call(
        _body,
        in_specs=[
            pl.BlockSpec(memory_space=pltpu.MemorySpace.VMEM),
            pl.BlockSpec(memory_space=pltpu.MemorySpace.VMEM),
        ],
        out_specs=pl.BlockSpec(memory_space=pl.ANY),
        out_shape=jax.ShapeDtypeStruct((bsz, 2 * f, h * w // 128, 128), jnp.float32),
        scratch_shapes=[
            pltpu.VMEM((f // 2, h * w // 128, 128), jnp.float32),
            pltpu.VMEM((f // 2, h * w // 128, 128), jnp.float32),
            pltpu.VMEM((f // 2, h * w // 128, 128), jnp.float32),
            pltpu.VMEM((f // 2, h * w // 128, 128), jnp.float32),
            pltpu.SemaphoreType.DMA((bsz, 4)),
        ],
    )(etop, ebot)
    return out.reshape(bsz, 2 * f, h, w)
